# Initial kernel scaffold; baseline (speedup 1.0000x reference)
import jax
import jax.numpy as jnp
from jax.experimental import pallas as pl


def kernel(f_subgraphs, phi_w1, phi_b1, phi_w2, phi_b2, rho_w1, rho_b1, rho_w2, rho_b2, segment_ids):
    raise NotImplementedError("write your pallas kernel here")


# speedup vs baseline: 2.7527x; 2.7527x over previous
import functools

import jax
import jax.numpy as jnp
from jax import lax
from jax.experimental import pallas as pl
from jax.experimental.pallas import tpu as pltpu
from jax.experimental.pallas import tpu_sc as plsc

N = 160000
D = 256
H = 256
M = 10000
OUT = 128
DH = 128

NC = 2
NS = 16
ROWS_PER_TILE = N // NS
CH = 80
NCH = ROWS_PER_TILE // CH
SEG_BLK = 1000
SEG_TILES = M // SEG_BLK

PHI_BLK = 1000
RHO_BLK = 1000


def _phi_body(x_ref, w1_ref, b1_ref, w2_ref, b2_ref, out_ref):
    x = x_ref[...]
    h = jnp.maximum(
        jnp.dot(x, w1_ref[...], preferred_element_type=jnp.float32)
        + b1_ref[...], 0.0)
    y = jnp.dot(h, w2_ref[...], preferred_element_type=jnp.float32) \
        + b2_ref[...]
    out_ref[0] = y[:, :DH]
    out_ref[1] = y[:, DH:]


def _phi_halves(x, w1, b1, w2, b2):
    grid = (N // PHI_BLK,)
    return pl.pallas_call(
        _phi_body,
        grid=grid,
        in_specs=[
            pl.BlockSpec((PHI_BLK, D), lambda i: (i, 0)),
            pl.BlockSpec((D, H), lambda i: (0, 0)),
            pl.BlockSpec((1, H), lambda i: (0, 0)),
            pl.BlockSpec((H, D), lambda i: (0, 0)),
            pl.BlockSpec((1, D), lambda i: (0, 0)),
        ],
        out_specs=pl.BlockSpec((NC, PHI_BLK, DH), lambda i: (0, i, 0)),
        out_shape=jax.ShapeDtypeStruct((NC, N, DH), jnp.float32),
        compiler_params=pltpu.CompilerParams(
            dimension_semantics=("arbitrary",)),
    )(x, w1, b1.reshape(1, H), w2, b2.reshape(1, D))


def _seg_body(phi_ref, ids_ref, zeros_ref, out_ref,
              acc, buf, ibuf, sr0, sr1, si0, si1):
    c = lax.axis_index("c")
    s = lax.axis_index("s")
    row0 = s * ROWS_PER_TILE
    srow = (sr0, sr1)
    sid = (si0, si1)

    def rd_descs(k, b):
        base = row0 + k * CH
        dr = pltpu.make_async_copy(
            phi_ref.at[c, pl.ds(base, CH)], buf.at[b], srow[b])
        di = pltpu.make_async_copy(
            ids_ref.at[pl.ds(base, CH)], ibuf.at[b], sid[b])
        return dr, di

    @pl.when(s < SEG_TILES)
    def _():
        pltpu.sync_copy(zeros_ref, acc.at[pl.ds(s * SEG_BLK, SEG_BLK)])
    plsc.subcore_barrier()

    for b in range(2):
        dr, di = rd_descs(b, b)
        dr.start()
        di.start()

    def step(k, b):
        dr, di = rd_descs(k, b)
        dr.wait()
        di.wait()
        pltpu.sync_copy(buf.at[b], acc.at[ibuf.at[b]], add=True)

        @pl.when(k + 2 < NCH)
        def _():
            dr2, di2 = rd_descs(k + 2, b)
            dr2.start()
            di2.start()

    @pl.loop(0, NCH - 1, step=2)
    def _(k0):
        step(k0, 0)
        step(k0 + 1, 1)

    step(NCH - 1, 0)

    plsc.subcore_barrier()

    @pl.when(s < SEG_TILES)
    def _():
        pltpu.sync_copy(acc.at[pl.ds(s * SEG_BLK, SEG_BLK)],
                        out_ref.at[c, pl.ds(s * SEG_BLK, SEG_BLK)])


def _segment_sum(phi_halves, ids):
    zeros = jnp.zeros((SEG_BLK, DH), jnp.float32)
    fn = pl.kernel(
        _seg_body,
        out_type=jax.ShapeDtypeStruct((NC, M, DH), jnp.float32),
        mesh=plsc.VectorSubcoreMesh(
            core_axis_name="c", subcore_axis_name="s",
            num_cores=NC, num_subcores=NS),
        scratch_types=[
            pltpu.VMEM_SHARED((M, DH), jnp.float32),
            pltpu.VMEM((2, CH, DH), jnp.float32),
            pltpu.VMEM((2, CH), jnp.int32),
            pltpu.SemaphoreType.DMA,
            pltpu.SemaphoreType.DMA,
            pltpu.SemaphoreType.DMA,
            pltpu.SemaphoreType.DMA,
        ],
    )
    return fn(phi_halves, ids, zeros)


def _rho_body(x_ref, w1_ref, b1_ref, w2_ref, b2_ref, out_ref):
    g = jnp.maximum(
        jnp.dot(x_ref[0], w1_ref[:DH, :], preferred_element_type=jnp.float32)
        + jnp.dot(x_ref[1], w1_ref[DH:, :], preferred_element_type=jnp.float32)
        + b1_ref[...], 0.0)
    out_ref[...] = jnp.dot(
        g, w2_ref[...], preferred_element_type=jnp.float32) + b2_ref[...]


def _rho(phi_mols, w1, b1, w2, b2):
    grid = (M // RHO_BLK,)
    return pl.pallas_call(
        _rho_body,
        grid=grid,
        in_specs=[
            pl.BlockSpec((NC, RHO_BLK, DH), lambda i: (0, i, 0)),
            pl.BlockSpec((D, H), lambda i: (0, 0)),
            pl.BlockSpec((1, H), lambda i: (0, 0)),
            pl.BlockSpec((H, OUT), lambda i: (0, 0)),
            pl.BlockSpec((1, OUT), lambda i: (0, 0)),
        ],
        out_specs=pl.BlockSpec((RHO_BLK, OUT), lambda i: (i, 0)),
        out_shape=jax.ShapeDtypeStruct((M, OUT), jnp.float32),
        compiler_params=pltpu.CompilerParams(
            dimension_semantics=("arbitrary",)),
    )(phi_mols, w1, b1.reshape(1, H), w2, b2.reshape(1, OUT))


@jax.jit
def kernel(f_subgraphs, phi_w1, phi_b1, phi_w2, phi_b2,
           rho_w1, rho_b1, rho_w2, rho_b2, segment_ids):
    phi_halves = _phi_halves(f_subgraphs, phi_w1, phi_b1, phi_w2, phi_b2)
    ids = segment_ids.astype(jnp.int32)
    phi_mols = _segment_sum(phi_halves, ids)
    return _rho(phi_mols, rho_w1, rho_b1, rho_w2, rho_b2)
